# baseline (speedup 1.0000x reference)
import functools

import jax
import jax.numpy as jnp
from jax.experimental import pallas as pl
from jax.experimental.pallas import tpu as pltpu

_H = 64
_W = 64
_HW = _H * _W
_VMEM = 56 * 1024 * 1024

_SHIFTS = tuple((dh, dw) for dh in (-1, 0, 1) for dw in (-1, 0, 1))


def _fc_kernel(x_ref, w_ref, b_ref, o_ref):
    o_ref[...] = (jnp.dot(x_ref[...], w_ref[...],
                          preferred_element_type=jnp.float32)
                  + b_ref[...]).astype(jnp.bfloat16)


def _fc(x, w_t, b2, *, tf=8192):
    N, Z = x.shape
    F = w_t.shape[1]
    assert F % tf == 0, (F, tf)
    return pl.pallas_call(
        _fc_kernel,
        out_shape=jax.ShapeDtypeStruct((N, F), jnp.bfloat16),
        grid=(F // tf,),
        in_specs=[pl.BlockSpec((N, Z), lambda j: (0, 0)),
                  pl.BlockSpec((Z, tf), lambda j: (0, j)),
                  pl.BlockSpec((1, tf), lambda j: (0, j))],
        out_specs=pl.BlockSpec((N, tf), lambda j: (0, j)),
        compiler_params=pltpu.CompilerParams(
            dimension_semantics=("parallel",),
            vmem_limit_bytes=_VMEM),
    )(x, w_t, b2)


def _tap_masks():
    pos = jax.lax.broadcasted_iota(jnp.int32, (1, _HW), 1)
    hh = pos // _W
    ww = pos - hh * _W
    masks = []
    for dh, dw in _SHIFTS:
        valid = ((hh >= -dh) & (hh < _H - dh) &
                 (ww >= -dw) & (ww < _W - dw))
        masks.append(valid)
    return masks


def _taps9(a, masks):
    parts = []
    for (dh, dw), m in zip(_SHIFTS, masks):
        delta = dh * _W + dw
        if delta == 0:
            parts.append(a)
        else:
            shifted = pltpu.roll(a, (-delta) % _HW, axis=1)
            parts.append(jnp.where(m, shifted, jnp.bfloat16(0)))
    return jnp.concatenate(parts, axis=0)


def _bn_coeffs(s_ref, ss_ref, g_ref, bt_ref, inv_cnt):
    mean = jnp.sum(s_ref[...], axis=0) * inv_cnt
    var = jnp.maximum(jnp.sum(ss_ref[...], axis=0) * inv_cnt - mean * mean,
                      0.0)
    scale = g_ref[...] * jax.lax.rsqrt(var + 1e-5)
    shift = bt_ref[...] - mean * scale
    return scale, shift


def _conv0_kernel(x_ref, w_ref, y_ref, so_ref, sso_ref):
    masks = _tap_masks()
    for b in range(x_ref.shape[0]):
        a = jnp.maximum(x_ref[b], jnp.bfloat16(0))
        y = jnp.dot(w_ref[...], _taps9(a, masks),
                    preferred_element_type=jnp.float32)
        y_ref[b] = y.astype(jnp.bfloat16)
        so_ref[b] = jnp.sum(y, axis=1, keepdims=True)
        sso_ref[b] = jnp.sum(y * y, axis=1, keepdims=True)


def _conv_kernel(x_ref, s_ref, ss_ref, g_ref, bt_ref, w_ref,
                 y_ref, so_ref, sso_ref, *, inv_cnt):
    scale, shift = _bn_coeffs(s_ref, ss_ref, g_ref, bt_ref, inv_cnt)
    masks = _tap_masks()
    for b in range(x_ref.shape[0]):
        a = jnp.maximum(x_ref[b].astype(jnp.float32) * scale + shift,
                        0.0).astype(jnp.bfloat16)
        y = jnp.dot(w_ref[...], _taps9(a, masks),
                    preferred_element_type=jnp.float32)
        y_ref[b] = y.astype(jnp.bfloat16)
        so_ref[b] = jnp.sum(y, axis=1, keepdims=True)
        sso_ref[b] = jnp.sum(y * y, axis=1, keepdims=True)


def _head_kernel(x_ref, s_ref, ss_ref, g_ref, bt_ref, w_ref,
                 mu_ref, lv_ref, *, inv_cnt):
    scale, shift = _bn_coeffs(s_ref, ss_ref, g_ref, bt_ref, inv_cnt)
    masks = _tap_masks()
    for b in range(x_ref.shape[0]):
        a = jnp.maximum(x_ref[b].astype(jnp.float32) * scale + shift,
                        0.0).astype(jnp.bfloat16)
        y = jnp.dot(w_ref[...], _taps9(a, masks),
                    preferred_element_type=jnp.float32)
        mu_ref[b] = y[:1]
        lv_ref[b] = y[1:2]


def _conv_block(x, w9, stats, *, first, B, inv_cnt):
    N, Cin, HW = x.shape
    Cout = w9.shape[0]
    grid = (N // B,)
    x_spec = pl.BlockSpec((B, Cin, HW), lambda n: (n, 0, 0))
    w_spec = pl.BlockSpec((Cout, 9 * Cin), lambda n: (0, 0))
    out_shape = (jax.ShapeDtypeStruct((N, Cout, HW), jnp.bfloat16),
                 jax.ShapeDtypeStruct((N, Cout, 1), jnp.float32),
                 jax.ShapeDtypeStruct((N, Cout, 1), jnp.float32))
    out_specs = (pl.BlockSpec((B, Cout, HW), lambda n: (n, 0, 0)),
                 pl.BlockSpec((B, Cout, 1), lambda n: (n, 0, 0)),
                 pl.BlockSpec((B, Cout, 1), lambda n: (n, 0, 0)))
    params = pltpu.CompilerParams(dimension_semantics=("parallel",),
                                  vmem_limit_bytes=_VMEM)
    if first:
        return pl.pallas_call(
            _conv0_kernel,
            out_shape=out_shape,
            grid=grid,
            in_specs=[x_spec, w_spec],
            out_specs=out_specs,
            compiler_params=params,
        )(x, w9)
    s, ss, g, bt = stats
    stat_spec = pl.BlockSpec((N, Cin, 1), lambda n: (0, 0, 0))
    vec_spec = pl.BlockSpec((Cin, 1), lambda n: (0, 0))
    return pl.pallas_call(
        functools.partial(_conv_kernel, inv_cnt=inv_cnt),
        out_shape=out_shape,
        grid=grid,
        in_specs=[x_spec, stat_spec, stat_spec, vec_spec, vec_spec, w_spec],
        out_specs=out_specs,
        compiler_params=params,
    )(x, s, ss, g, bt, w9)


def _head_block(x, w9, stats, *, B, inv_cnt):
    N, Cin, HW = x.shape
    s, ss, g, bt = stats
    out_shape = (jax.ShapeDtypeStruct((N, 1, HW), jnp.float32),
                 jax.ShapeDtypeStruct((N, 1, HW), jnp.float32))
    o_spec = pl.BlockSpec((B, 1, HW), lambda n: (n, 0, 0))
    return pl.pallas_call(
        functools.partial(_head_kernel, inv_cnt=inv_cnt),
        out_shape=out_shape,
        grid=(N // B,),
        in_specs=[pl.BlockSpec((B, Cin, HW), lambda n: (n, 0, 0)),
                  pl.BlockSpec((N, Cin, 1), lambda n: (0, 0, 0)),
                  pl.BlockSpec((N, Cin, 1), lambda n: (0, 0, 0)),
                  pl.BlockSpec((Cin, 1), lambda n: (0, 0)),
                  pl.BlockSpec((Cin, 1), lambda n: (0, 0)),
                  pl.BlockSpec((w9.shape[0], 9 * Cin), lambda n: (0, 0))],
        out_specs=(o_spec, o_spec),
        compiler_params=pltpu.CompilerParams(
            dimension_semantics=("parallel",),
            vmem_limit_bytes=_VMEM),
    )(x, s, ss, g, bt, w9)


def _w9(w_taps):
    return jnp.concatenate([w_taps[0], w_taps[1], w_taps[2]],
                           axis=1).astype(jnp.bfloat16)


def kernel(x, fc_w_t, fc_b, w0, gamma0, beta0, w1, gamma1, beta1,
           w2, gamma2, beta2, w_head):
    N = x.shape[0]
    B = 4 if N % 4 == 0 else 1
    inv_cnt = 1.0 / float(N * _HW)
    c0 = 48

    fc = _fc(x, fc_w_t, fc_b.reshape(1, -1))
    cur = fc.reshape(N, c0, _HW)

    y, s, ss = _conv_block(cur, _w9(w0), None, first=True, B=B,
                           inv_cnt=inv_cnt)
    for w, g, bt in ((w1, gamma0, beta0), (w2, gamma1, beta1)):
        stats = (s, ss, g[:, None], bt[:, None])
        y, s, ss = _conv_block(y, _w9(w), stats, first=False, B=B,
                               inv_cnt=inv_cnt)
    stats = (s, ss, gamma2[:, None], beta2[:, None])
    mu, lv = _head_block(y, _w9(w_head), stats, B=B, inv_cnt=inv_cnt)
    return (mu.reshape(N, 1, _H, _W), lv.reshape(N, 1, _H, _W))


# speedup vs baseline: 1.0152x; 1.0152x over previous
import functools

import jax
import jax.numpy as jnp
from jax.experimental import pallas as pl
from jax.experimental.pallas import tpu as pltpu

_H = 64
_W = 64
_HW = _H * _W
_VMEM = 56 * 1024 * 1024

_SHIFTS = tuple((dh, dw) for dh in (-1, 0, 1) for dw in (-1, 0, 1))


def _fc_kernel(x_ref, w_ref, b_ref, o_ref):
    o_ref[...] = (jnp.dot(x_ref[...], w_ref[...],
                          preferred_element_type=jnp.float32)
                  + b_ref[...]).astype(jnp.bfloat16)


def _fc(x, w_t, b2, *, tf=16384):
    N, Z = x.shape
    F = w_t.shape[1]
    assert F % tf == 0, (F, tf)
    return pl.pallas_call(
        _fc_kernel,
        out_shape=jax.ShapeDtypeStruct((N, F), jnp.bfloat16),
        grid=(F // tf,),
        in_specs=[pl.BlockSpec((N, Z), lambda j: (0, 0)),
                  pl.BlockSpec((Z, tf), lambda j: (0, j)),
                  pl.BlockSpec((1, tf), lambda j: (0, j))],
        out_specs=pl.BlockSpec((N, tf), lambda j: (0, j)),
        compiler_params=pltpu.CompilerParams(
            dimension_semantics=("parallel",),
            vmem_limit_bytes=_VMEM),
    )(x, w_t, b2)


def _tap_masks():
    pos = jax.lax.broadcasted_iota(jnp.int32, (1, _HW), 1)
    hh = pos // _W
    ww = pos - hh * _W
    masks = []
    for dh, dw in _SHIFTS:
        valid = ((hh >= -dh) & (hh < _H - dh) &
                 (ww >= -dw) & (ww < _W - dw))
        masks.append(valid)
    return masks


def _taps9(a, masks):
    parts = []
    for (dh, dw), m in zip(_SHIFTS, masks):
        delta = dh * _W + dw
        if delta == 0:
            parts.append(a)
        else:
            shifted = pltpu.roll(a, (-delta) % _HW, axis=1)
            parts.append(jnp.where(m, shifted, jnp.bfloat16(0)))
    return jnp.concatenate(parts, axis=0)


def _bn_coeffs(s_ref, ss_ref, g_ref, bt_ref, inv_cnt):
    mean = jnp.sum(s_ref[...], axis=0) * inv_cnt
    var = jnp.maximum(jnp.sum(ss_ref[...], axis=0) * inv_cnt - mean * mean,
                      0.0)
    scale = g_ref[...] * jax.lax.rsqrt(var + 1e-5)
    shift = bt_ref[...] - mean * scale
    return scale, shift


def _conv0_kernel(x_ref, w_ref, y_ref, so_ref, sso_ref):
    masks = _tap_masks()
    for b in range(x_ref.shape[0]):
        a = jnp.maximum(x_ref[b], jnp.bfloat16(0))
        y = jnp.dot(w_ref[...], _taps9(a, masks),
                    preferred_element_type=jnp.float32)
        y_ref[b] = y.astype(jnp.bfloat16)
        so_ref[b] = jnp.sum(y, axis=1, keepdims=True)
        sso_ref[b] = jnp.sum(y * y, axis=1, keepdims=True)


def _conv_kernel(x_ref, s_ref, ss_ref, g_ref, bt_ref, w_ref,
                 y_ref, so_ref, sso_ref, *, inv_cnt):
    scale, shift = _bn_coeffs(s_ref, ss_ref, g_ref, bt_ref, inv_cnt)
    masks = _tap_masks()
    for b in range(x_ref.shape[0]):
        a = jnp.maximum(x_ref[b].astype(jnp.float32) * scale + shift,
                        0.0).astype(jnp.bfloat16)
        y = jnp.dot(w_ref[...], _taps9(a, masks),
                    preferred_element_type=jnp.float32)
        y_ref[b] = y.astype(jnp.bfloat16)
        so_ref[b] = jnp.sum(y, axis=1, keepdims=True)
        sso_ref[b] = jnp.sum(y * y, axis=1, keepdims=True)


def _head_kernel(x_ref, s_ref, ss_ref, g_ref, bt_ref, w_ref,
                 mu_ref, lv_ref, *, inv_cnt):
    scale, shift = _bn_coeffs(s_ref, ss_ref, g_ref, bt_ref, inv_cnt)
    masks = _tap_masks()
    for b in range(x_ref.shape[0]):
        a = jnp.maximum(x_ref[b].astype(jnp.float32) * scale + shift,
                        0.0).astype(jnp.bfloat16)
        y = jnp.dot(w_ref[...], _taps9(a, masks),
                    preferred_element_type=jnp.float32)
        mu_ref[b] = y[:1]
        lv_ref[b] = y[1:2]


def _conv_block(x, w9, stats, *, first, B, inv_cnt):
    N, Cin, HW = x.shape
    Cout = w9.shape[0]
    grid = (N // B,)
    x_spec = pl.BlockSpec((B, Cin, HW), lambda n: (n, 0, 0))
    w_spec = pl.BlockSpec((Cout, 9 * Cin), lambda n: (0, 0))
    out_shape = (jax.ShapeDtypeStruct((N, Cout, HW), jnp.bfloat16),
                 jax.ShapeDtypeStruct((N, Cout, 1), jnp.float32),
                 jax.ShapeDtypeStruct((N, Cout, 1), jnp.float32))
    out_specs = (pl.BlockSpec((B, Cout, HW), lambda n: (n, 0, 0)),
                 pl.BlockSpec((B, Cout, 1), lambda n: (n, 0, 0)),
                 pl.BlockSpec((B, Cout, 1), lambda n: (n, 0, 0)))
    params = pltpu.CompilerParams(dimension_semantics=("parallel",),
                                  vmem_limit_bytes=_VMEM)
    if first:
        return pl.pallas_call(
            _conv0_kernel,
            out_shape=out_shape,
            grid=grid,
            in_specs=[x_spec, w_spec],
            out_specs=out_specs,
            compiler_params=params,
        )(x, w9)
    s, ss, g, bt = stats
    stat_spec = pl.BlockSpec((N, Cin, 1), lambda n: (0, 0, 0))
    vec_spec = pl.BlockSpec((Cin, 1), lambda n: (0, 0))
    return pl.pallas_call(
        functools.partial(_conv_kernel, inv_cnt=inv_cnt),
        out_shape=out_shape,
        grid=grid,
        in_specs=[x_spec, stat_spec, stat_spec, vec_spec, vec_spec, w_spec],
        out_specs=out_specs,
        compiler_params=params,
    )(x, s, ss, g, bt, w9)


def _head_block(x, w9, stats, *, B, inv_cnt):
    N, Cin, HW = x.shape
    s, ss, g, bt = stats
    out_shape = (jax.ShapeDtypeStruct((N, 1, HW), jnp.float32),
                 jax.ShapeDtypeStruct((N, 1, HW), jnp.float32))
    o_spec = pl.BlockSpec((B, 1, HW), lambda n: (n, 0, 0))
    return pl.pallas_call(
        functools.partial(_head_kernel, inv_cnt=inv_cnt),
        out_shape=out_shape,
        grid=(N // B,),
        in_specs=[pl.BlockSpec((B, Cin, HW), lambda n: (n, 0, 0)),
                  pl.BlockSpec((N, Cin, 1), lambda n: (0, 0, 0)),
                  pl.BlockSpec((N, Cin, 1), lambda n: (0, 0, 0)),
                  pl.BlockSpec((Cin, 1), lambda n: (0, 0)),
                  pl.BlockSpec((Cin, 1), lambda n: (0, 0)),
                  pl.BlockSpec((w9.shape[0], 9 * Cin), lambda n: (0, 0))],
        out_specs=(o_spec, o_spec),
        compiler_params=pltpu.CompilerParams(
            dimension_semantics=("parallel",),
            vmem_limit_bytes=_VMEM),
    )(x, s, ss, g, bt, w9)


def _w9(w_taps):
    return jnp.concatenate([w_taps[0], w_taps[1], w_taps[2]],
                           axis=1).astype(jnp.bfloat16)


def kernel(x, fc_w_t, fc_b, w0, gamma0, beta0, w1, gamma1, beta1,
           w2, gamma2, beta2, w_head):
    N = x.shape[0]
    B = 4 if N % 4 == 0 else 1
    inv_cnt = 1.0 / float(N * _HW)
    c0 = 48

    fc = _fc(x, fc_w_t, fc_b.reshape(1, -1))
    cur = fc.reshape(N, c0, _HW)

    y, s, ss = _conv_block(cur, _w9(w0), None, first=True, B=B,
                           inv_cnt=inv_cnt)
    for w, g, bt in ((w1, gamma0, beta0), (w2, gamma1, beta1)):
        stats = (s, ss, g[:, None], bt[:, None])
        y, s, ss = _conv_block(y, _w9(w), stats, first=False, B=B,
                               inv_cnt=inv_cnt)
    stats = (s, ss, gamma2[:, None], beta2[:, None])
    mu, lv = _head_block(y, _w9(w_head), stats, B=B, inv_cnt=inv_cnt)
    return (mu.reshape(N, 1, _H, _W), lv.reshape(N, 1, _H, _W))
